# baseline (speedup 1.0000x reference)
import functools

import jax
import jax.numpy as jnp
import numpy as np
from jax import lax
from jax.experimental import pallas as pl
from jax.experimental.pallas import tpu as pltpu
from jax.experimental.pallas import tpu_sc as plsc

_R = 128
_N = 32768
_K = 256
_L = 16
_NC = 2
_NS = 16
_NW = _NC * _NS
_ROWS_PER_W = _R // _NW
_SLICES = _N // _L
_UNROLL = 16
_NB = 256
_TOPBIT = np.int32(-(2 ** 31))
_MAXPOS = np.int32(0x7FFFFFFF)

_FILTER = np.float32(0.75)
_MAGNIFY = np.float32(1.25)


def _rank_body(y_hbm, out_hbm, buf0, buf1, buf2, hist_v, merged_v,
               in_sem, out_sem):
    cid = lax.axis_index("c")
    sid = lax.axis_index("s")
    wid = sid * _NC + cid
    row0 = wid * _ROWS_PER_W

    iota = lax.iota(jnp.int32, _L)
    zero_v = iota * np.int32(0)
    ones_v = zero_v + np.int32(1)
    bufs = [buf0, buf1, buf2]

    @plsc.parallel_loop(0, (_L * _NB) // _L, 1, unroll=_UNROLL)
    def zero_pass(i):
        hist_v[pl.ds(i * _L, _L)] = zero_v

    pltpu.async_copy(y_hbm.at[row0], buf0, in_sem)

    for r in range(_ROWS_PER_W):
        b = bufs[r % 3]
        nxt = bufs[(r + 1) % 3]
        if r + 1 < _ROWS_PER_W:
            if r - 2 >= 0:
                pltpu.make_async_copy(
                    nxt, out_hbm.at[row0 + r - 2], out_sem).wait()
            pltpu.async_copy(y_hbm.at[row0 + r + 1], nxt, in_sem)
        pltpu.make_async_copy(y_hbm.at[row0 + r], b, in_sem).wait()

        @plsc.parallel_loop(0, _SLICES, 1, unroll=_UNROLL)
        def hist0_pass(i):
            off = i * _L
            bi = plsc.bitcast(b[pl.ds(off, _L)], jnp.int32)
            ku = bi ^ ((bi >> 31) | _TOPBIT)
            b16 = lax.shift_right_logical(ku, np.int32(20)) & np.int32(0xFF0)
            plsc.addupdate_scatter(hist_v, [b16 | iota], ones_v)

        prefix = wid * np.int32(0)
        k_rem = prefix + np.int32(_K)
        for p in range(4):
            shift = 24 - 8 * p
            if p > 0:
                _shift, _pref = shift, prefix

                @plsc.parallel_loop(0, _SLICES, 1, unroll=_UNROLL)
                def hist_pass(i):
                    bi = plsc.bitcast(b[pl.ds(i * _L, _L)], jnp.int32)
                    ku = bi ^ ((bi >> 31) | _TOPBIT)
                    active = lax.shift_right_logical(
                        ku ^ _pref, np.int32(_shift + 8)) == 0
                    if _shift >= 4:
                        b16 = lax.shift_right_logical(
                            ku, np.int32(_shift - 4)) & np.int32(0xFF0)
                    else:
                        b16 = lax.shift_left(
                            ku, np.int32(4 - _shift)) & np.int32(0xFF0)
                    plsc.addupdate_scatter(
                        hist_v, [b16 | iota], ones_v, mask=active)

            @plsc.parallel_loop(0, _L, 1, unroll=2)
            def merge_pass(c):
                acc = zero_v
                for q in range(_L):
                    acc = acc + hist_v[pl.ds(c * _NB + q * _L, _L)]
                tot = jnp.sum(acc)
                plsc.store_scatter(merged_v, [zero_v + c], zero_v + tot,
                                   mask=iota == 0)

            csums = merged_v[pl.ds(0, _L)]

            cs = plsc.cumsum(csums)
            tot_all = jnp.sum(csums)
            hi = tot_all - cs + csums
            m = (hi >= k_rem) & (hi - csums < k_rem)
            mi = m.astype(jnp.int32)
            chunk = jnp.sum(mi * iota)
            above = jnp.sum(jnp.where(m, hi - csums, zero_v))

            @plsc.parallel_loop(0, _L, 1, unroll=4)
            def within_pass(j):
                vb = hist_v[pl.ds((chunk * _L + j) * _L, _L)]
                tb = jnp.sum(vb)
                plsc.store_scatter(merged_v, [zero_v + j], zero_v + tb,
                                   mask=iota == 0)

            tot_c = merged_v[pl.ds(0, _L)]
            cs2 = plsc.cumsum(tot_c)
            hi2 = above + (jnp.sum(tot_c) - cs2 + tot_c)
            m2 = (hi2 >= k_rem) & (hi2 - tot_c < k_rem)
            mi2 = m2.astype(jnp.int32)
            b_loc = jnp.sum(mi2 * iota)
            sub = jnp.sum(jnp.where(m2, hi2 - tot_c, zero_v))

            prefix = prefix | lax.shift_left(chunk * _L + b_loc,
                                             np.int32(shift))
            k_rem = k_rem - sub

            @plsc.parallel_loop(0, (_L * _NB) // _L, 1, unroll=_UNROLL)
            def rezero_pass(i):
                hist_v[pl.ds(i * _L, _L)] = zero_v

        t_bits = jnp.where(prefix < 0, prefix ^ _TOPBIT, ~prefix)
        t_f = plsc.bitcast(zero_v + t_bits, jnp.float32)

        @plsc.parallel_loop(0, _SLICES, 1, unroll=_UNROLL)
        def scale_pass(i):
            off = i * _L
            x = b[pl.ds(off, _L)]
            b[pl.ds(off, _L)] = jnp.where(x < t_f, x * _FILTER, x * _MAGNIFY)

        pltpu.async_copy(b, out_hbm.at[row0 + r], out_sem)

    for r in range(max(0, _ROWS_PER_W - 3), _ROWS_PER_W):
        pltpu.make_async_copy(
            bufs[r % 3], out_hbm.at[row0 + r], out_sem).wait()


_rank_sc = functools.partial(
    pl.kernel,
    out_type=jax.ShapeDtypeStruct((_R, _N), jnp.float32),
    mesh=plsc.VectorSubcoreMesh(core_axis_name="c", subcore_axis_name="s"),
    scratch_types=[
        pltpu.VMEM((_N,), jnp.float32),
        pltpu.VMEM((_N,), jnp.float32),
        pltpu.VMEM((_N,), jnp.float32),
        pltpu.VMEM((_L * _NB,), jnp.int32),
        pltpu.VMEM((_L,), jnp.int32),
        pltpu.SemaphoreType.DMA,
        pltpu.SemaphoreType.DMA,
    ],
    compiler_params=pltpu.CompilerParams(needs_layout_passes=False),
)(_rank_body)


def kernel(y):
    return _rank_sc(y)


# speedup vs baseline: 1.1064x; 1.1064x over previous
import functools

import jax
import jax.numpy as jnp
import numpy as np
from jax import lax
from jax.experimental import pallas as pl
from jax.experimental.pallas import tpu as pltpu
from jax.experimental.pallas import tpu_sc as plsc

_R = 128
_N = 32768
_K = 256
_L = 16
_NC = 2
_NS = 16
_NW = _NC * _NS
_ROWS_PER_W = _R // _NW
_SLICES = _N // _L
_UNROLL = 16
_NB = 256
_CAP = 4096
_TOPBIT = np.int32(-(2 ** 31))
_MAXPOS = np.int32(0x7FFFFFFF)

_FILTER = np.float32(0.75)
_MAGNIFY = np.float32(1.25)


def _rank_body(y_hbm, out_hbm, buf0, buf1, buf2, hist_v, merged_v, comp_v,
               in_sem, out_sem):
    cid = lax.axis_index("c")
    sid = lax.axis_index("s")
    wid = sid * _NC + cid
    row0 = wid * _ROWS_PER_W

    iota = lax.iota(jnp.int32, _L)
    zero_v = iota * np.int32(0)
    ones_v = zero_v + np.int32(1)
    bufs = [buf0, buf1, buf2]

    @plsc.parallel_loop(0, (_L * _NB) // _L, 1, unroll=_UNROLL)
    def zero_pass(i):
        hist_v[pl.ds(i * _L, _L)] = zero_v

    pltpu.async_copy(y_hbm.at[row0], buf0, in_sem)

    for r in range(_ROWS_PER_W):
        b = bufs[r % 3]
        nxt = bufs[(r + 1) % 3]
        if r + 1 < _ROWS_PER_W:
            if r - 2 >= 0:
                pltpu.make_async_copy(
                    nxt, out_hbm.at[row0 + r - 2], out_sem).wait()
            pltpu.async_copy(y_hbm.at[row0 + r + 1], nxt, in_sem)
        pltpu.make_async_copy(y_hbm.at[row0 + r], b, in_sem).wait()

        @plsc.parallel_loop(0, _SLICES, 1, unroll=_UNROLL)
        def hist0_pass(i):
            off = i * _L
            bi = plsc.bitcast(b[pl.ds(off, _L)], jnp.int32)
            ku = bi ^ ((bi >> 31) | _TOPBIT)
            b16 = lax.shift_right_logical(ku, np.int32(20)) & np.int32(0xFF0)
            plsc.addupdate_scatter(hist_v, [b16 | iota], ones_v)

        prefix = wid * np.int32(0)
        k_rem = prefix + np.int32(_K)
        n1 = prefix
        for p in range(4):
            shift = 24 - 8 * p
            if p == 1:
                b0_splat = zero_v + lax.shift_right_logical(prefix, 24)
                n1_splat = zero_v + n1
                cap_ok = n1 <= np.int32(_CAP)

                @plsc.parallel_loop(0, _SLICES, 1, unroll=8,
                                    carry=zero_v)
                def compact_pass(i, off):
                    x = b[pl.ds(i * _L, _L)]
                    bi = plsc.bitcast(x, jnp.int32)
                    ku = bi ^ ((bi >> 31) | _TOPBIT)
                    active = lax.shift_right_logical(
                        ku, np.int32(24)) == b0_splat
                    pos = off + plsc.cumsum(active.astype(jnp.int32))
                    ok = active & (pos <= np.int32(_CAP))
                    plsc.store_scatter(comp_v, [pos - 1], x, mask=ok)
                    cnt = plsc.all_reduce_population_count(active)
                    return off + cnt

            if p > 0:
                _shift, _pref = shift, prefix
                nslices = lax.div(n1 + np.int32(_L - 1), np.int32(_L))

                @pl.when(cap_ok)
                def _small_hist():
                    def body(i, carry):
                        x = comp_v[pl.ds(i * _L, _L)]
                        bi = plsc.bitcast(x, jnp.int32)
                        ku = bi ^ ((bi >> 31) | _TOPBIT)
                        lane_ok = (i * _L + iota) < n1_splat
                        active = (lax.shift_right_logical(
                            ku ^ _pref, np.int32(_shift + 8)) == 0) & lane_ok
                        if _shift >= 4:
                            b16 = lax.shift_right_logical(
                                ku, np.int32(_shift - 4)) & np.int32(0xFF0)
                        else:
                            b16 = lax.shift_left(
                                ku, np.int32(4 - _shift)) & np.int32(0xFF0)
                        plsc.addupdate_scatter(
                            hist_v, [b16 | iota], ones_v, mask=active)
                        return carry
                    lax.fori_loop(0, nslices, body, np.int32(0))

                @pl.when(jnp.logical_not(cap_ok))
                def _full_hist():
                    @plsc.parallel_loop(0, _SLICES, 1, unroll=_UNROLL)
                    def hist_pass(i):
                        bi = plsc.bitcast(b[pl.ds(i * _L, _L)], jnp.int32)
                        ku = bi ^ ((bi >> 31) | _TOPBIT)
                        active = lax.shift_right_logical(
                            ku ^ _pref, np.int32(_shift + 8)) == 0
                        if _shift >= 4:
                            b16 = lax.shift_right_logical(
                                ku, np.int32(_shift - 4)) & np.int32(0xFF0)
                        else:
                            b16 = lax.shift_left(
                                ku, np.int32(4 - _shift)) & np.int32(0xFF0)
                        plsc.addupdate_scatter(
                            hist_v, [b16 | iota], ones_v, mask=active)

            @plsc.parallel_loop(0, _L, 1, unroll=2)
            def merge_pass(c):
                acc = zero_v
                for q in range(_L):
                    acc = acc + hist_v[pl.ds(c * _NB + q * _L, _L)]
                tot = jnp.sum(acc)
                plsc.store_scatter(merged_v, [zero_v + c], zero_v + tot,
                                   mask=iota == 0)

            csums = merged_v[pl.ds(0, _L)]

            cs = plsc.cumsum(csums)
            tot_all = jnp.sum(csums)
            hi = tot_all - cs + csums
            m = (hi >= k_rem) & (hi - csums < k_rem)
            mi = m.astype(jnp.int32)
            chunk = jnp.sum(mi * iota)
            above = jnp.sum(jnp.where(m, hi - csums, zero_v))

            @plsc.parallel_loop(0, _L, 1, unroll=4)
            def within_pass(j):
                vb = hist_v[pl.ds((chunk * _L + j) * _L, _L)]
                tb = jnp.sum(vb)
                plsc.store_scatter(merged_v, [zero_v + j], zero_v + tb,
                                   mask=iota == 0)

            tot_c = merged_v[pl.ds(0, _L)]
            cs2 = plsc.cumsum(tot_c)
            hi2 = above + (jnp.sum(tot_c) - cs2 + tot_c)
            m2 = (hi2 >= k_rem) & (hi2 - tot_c < k_rem)
            mi2 = m2.astype(jnp.int32)
            b_loc = jnp.sum(mi2 * iota)
            sub = jnp.sum(jnp.where(m2, hi2 - tot_c, zero_v))

            prefix = prefix | lax.shift_left(chunk * _L + b_loc,
                                             np.int32(shift))
            k_rem = k_rem - sub
            if p == 0:
                n1 = jnp.sum(jnp.where(m2, tot_c, zero_v))

            @plsc.parallel_loop(0, (_L * _NB) // _L, 1, unroll=_UNROLL)
            def rezero_pass(i):
                hist_v[pl.ds(i * _L, _L)] = zero_v

        t_bits = jnp.where(prefix < 0, prefix ^ _TOPBIT, ~prefix)
        t_f = plsc.bitcast(zero_v + t_bits, jnp.float32)

        @plsc.parallel_loop(0, _SLICES, 1, unroll=_UNROLL)
        def scale_pass(i):
            off = i * _L
            x = b[pl.ds(off, _L)]
            b[pl.ds(off, _L)] = jnp.where(x < t_f, x * _FILTER, x * _MAGNIFY)

        pltpu.async_copy(b, out_hbm.at[row0 + r], out_sem)

    for r in range(max(0, _ROWS_PER_W - 3), _ROWS_PER_W):
        pltpu.make_async_copy(
            bufs[r % 3], out_hbm.at[row0 + r], out_sem).wait()


_rank_sc = functools.partial(
    pl.kernel,
    out_type=jax.ShapeDtypeStruct((_R, _N), jnp.float32),
    mesh=plsc.VectorSubcoreMesh(core_axis_name="c", subcore_axis_name="s"),
    scratch_types=[
        pltpu.VMEM((_N,), jnp.float32),
        pltpu.VMEM((_N,), jnp.float32),
        pltpu.VMEM((_N,), jnp.float32),
        pltpu.VMEM((_L * _NB,), jnp.int32),
        pltpu.VMEM((_L,), jnp.int32),
        pltpu.VMEM((_CAP,), jnp.float32),
        pltpu.SemaphoreType.DMA,
        pltpu.SemaphoreType.DMA,
    ],
    compiler_params=pltpu.CompilerParams(needs_layout_passes=False),
)(_rank_body)


def kernel(y):
    return _rank_sc(y)


# speedup vs baseline: 1.1500x; 1.0394x over previous
import functools

import jax
import jax.numpy as jnp
import numpy as np
from jax import lax
from jax.experimental import pallas as pl
from jax.experimental.pallas import tpu as pltpu
from jax.experimental.pallas import tpu_sc as plsc

_R = 128
_N = 32768
_K = 256
_L = 16
_NC = 2
_NS = 16
_NW = _NC * _NS
_ROWS_PER_W = _R // _NW
_SLICES = _N // _L
_UNROLL = 16
_NB = 256
_CAP = 4096
_TOPBIT = np.int32(-(2 ** 31))
_MAXPOS = np.int32(0x7FFFFFFF)

_FILTER = np.float32(0.75)
_MAGNIFY = np.float32(1.25)


def _compact_step(b, comp_v, i, cnt, b0_splat, iota):
    x = b[pl.ds(i * _L, _L)]
    bi = plsc.bitcast(x, jnp.int32)
    ku = bi ^ ((bi >> 31) | _TOPBIT)
    active = lax.shift_right_logical(ku, np.int32(24)) == b0_splat
    ok = active & (cnt < np.int32(_CAP // _L))
    idx = lax.shift_left(cnt, np.int32(4)) | iota
    plsc.store_scatter(comp_v, [idx], x, mask=ok)
    return cnt + active.astype(jnp.int32)


def _rank_body(y_hbm, out_hbm, buf0, buf1, buf2, hist_v, merged_v, comp_v,
               in_sem, out_sem):
    cid = lax.axis_index("c")
    sid = lax.axis_index("s")
    wid = sid * _NC + cid
    row0 = wid * _ROWS_PER_W

    iota = lax.iota(jnp.int32, _L)
    zero_v = iota * np.int32(0)
    ones_v = zero_v + np.int32(1)
    bufs = [buf0, buf1, buf2]

    @plsc.parallel_loop(0, (_L * _NB) // _L, 1, unroll=_UNROLL)
    def zero_pass(i):
        hist_v[pl.ds(i * _L, _L)] = zero_v

    pltpu.async_copy(y_hbm.at[row0], buf0, in_sem)

    for r in range(_ROWS_PER_W):
        b = bufs[r % 3]
        nxt = bufs[(r + 1) % 3]
        if r + 1 < _ROWS_PER_W:
            if r - 2 >= 0:
                pltpu.make_async_copy(
                    nxt, out_hbm.at[row0 + r - 2], out_sem).wait()
            pltpu.async_copy(y_hbm.at[row0 + r + 1], nxt, in_sem)
        pltpu.make_async_copy(y_hbm.at[row0 + r], b, in_sem).wait()

        @plsc.parallel_loop(0, _SLICES, 1, unroll=_UNROLL)
        def hist0_pass(i):
            off = i * _L
            bi = plsc.bitcast(b[pl.ds(off, _L)], jnp.int32)
            ku = bi ^ ((bi >> 31) | _TOPBIT)
            b16 = lax.shift_right_logical(ku, np.int32(20)) & np.int32(0xFF0)
            plsc.addupdate_scatter(hist_v, [b16 | iota], ones_v)

        prefix = wid * np.int32(0)
        k_rem = prefix + np.int32(_K)
        n1 = prefix
        for p in range(4):
            shift = 24 - 8 * p
            if p == 1:
                b0_splat = zero_v + lax.shift_right_logical(prefix, 24)

                cnts = plsc.parallel_loop(
                    0, _SLICES, 1, unroll=8, carry=zero_v)(
                    lambda i, cnt: _compact_step(b, comp_v, i, cnt,
                                                 b0_splat, iota))
                maxcnt = jnp.max(cnts)
                cap_ok = maxcnt <= np.int32(_CAP // _L)

            if p > 0:
                _shift, _pref = shift, prefix

                @pl.when(cap_ok)
                def _small_hist():
                    def body(s, carry):
                        x = comp_v[pl.ds(s * _L, _L)]
                        bi = plsc.bitcast(x, jnp.int32)
                        ku = bi ^ ((bi >> 31) | _TOPBIT)
                        valid = cnts > s
                        active = (lax.shift_right_logical(
                            ku ^ _pref, np.int32(_shift + 8)) == 0) & valid
                        if _shift >= 4:
                            b16 = lax.shift_right_logical(
                                ku, np.int32(_shift - 4)) & np.int32(0xFF0)
                        else:
                            b16 = lax.shift_left(
                                ku, np.int32(4 - _shift)) & np.int32(0xFF0)
                        plsc.addupdate_scatter(
                            hist_v, [b16 | iota], ones_v, mask=active)
                        return carry
                    lax.fori_loop(0, maxcnt, body, np.int32(0))

                @pl.when(jnp.logical_not(cap_ok))
                def _full_hist():
                    @plsc.parallel_loop(0, _SLICES, 1, unroll=_UNROLL)
                    def hist_pass(i):
                        bi = plsc.bitcast(b[pl.ds(i * _L, _L)], jnp.int32)
                        ku = bi ^ ((bi >> 31) | _TOPBIT)
                        active = lax.shift_right_logical(
                            ku ^ _pref, np.int32(_shift + 8)) == 0
                        if _shift >= 4:
                            b16 = lax.shift_right_logical(
                                ku, np.int32(_shift - 4)) & np.int32(0xFF0)
                        else:
                            b16 = lax.shift_left(
                                ku, np.int32(4 - _shift)) & np.int32(0xFF0)
                        plsc.addupdate_scatter(
                            hist_v, [b16 | iota], ones_v, mask=active)

            @plsc.parallel_loop(0, _L, 1, unroll=2)
            def merge_pass(c):
                acc = zero_v
                for q in range(_L):
                    acc = acc + hist_v[pl.ds(c * _NB + q * _L, _L)]
                tot = jnp.sum(acc)
                plsc.store_scatter(merged_v, [zero_v + c], zero_v + tot,
                                   mask=iota == 0)

            csums = merged_v[pl.ds(0, _L)]

            cs = plsc.cumsum(csums)
            tot_all = jnp.sum(csums)
            hi = tot_all - cs + csums
            m = (hi >= k_rem) & (hi - csums < k_rem)
            mi = m.astype(jnp.int32)
            chunk = jnp.sum(mi * iota)
            above = jnp.sum(jnp.where(m, hi - csums, zero_v))

            @plsc.parallel_loop(0, _L, 1, unroll=4)
            def within_pass(j):
                vb = hist_v[pl.ds((chunk * _L + j) * _L, _L)]
                tb = jnp.sum(vb)
                plsc.store_scatter(merged_v, [zero_v + j], zero_v + tb,
                                   mask=iota == 0)

            tot_c = merged_v[pl.ds(0, _L)]
            cs2 = plsc.cumsum(tot_c)
            hi2 = above + (jnp.sum(tot_c) - cs2 + tot_c)
            m2 = (hi2 >= k_rem) & (hi2 - tot_c < k_rem)
            mi2 = m2.astype(jnp.int32)
            b_loc = jnp.sum(mi2 * iota)
            sub = jnp.sum(jnp.where(m2, hi2 - tot_c, zero_v))

            prefix = prefix | lax.shift_left(chunk * _L + b_loc,
                                             np.int32(shift))
            k_rem = k_rem - sub
            if p == 0:
                n1 = jnp.sum(jnp.where(m2, tot_c, zero_v))

            @plsc.parallel_loop(0, (_L * _NB) // _L, 1, unroll=_UNROLL)
            def rezero_pass(i):
                hist_v[pl.ds(i * _L, _L)] = zero_v

        t_bits = jnp.where(prefix < 0, prefix ^ _TOPBIT, ~prefix)
        t_f = plsc.bitcast(zero_v + t_bits, jnp.float32)

        @plsc.parallel_loop(0, _SLICES, 1, unroll=_UNROLL)
        def scale_pass(i):
            off = i * _L
            x = b[pl.ds(off, _L)]
            b[pl.ds(off, _L)] = jnp.where(x < t_f, x * _FILTER, x * _MAGNIFY)

        pltpu.async_copy(b, out_hbm.at[row0 + r], out_sem)

    for r in range(max(0, _ROWS_PER_W - 3), _ROWS_PER_W):
        pltpu.make_async_copy(
            bufs[r % 3], out_hbm.at[row0 + r], out_sem).wait()


_rank_sc = functools.partial(
    pl.kernel,
    out_type=jax.ShapeDtypeStruct((_R, _N), jnp.float32),
    mesh=plsc.VectorSubcoreMesh(core_axis_name="c", subcore_axis_name="s"),
    scratch_types=[
        pltpu.VMEM((_N,), jnp.float32),
        pltpu.VMEM((_N,), jnp.float32),
        pltpu.VMEM((_N,), jnp.float32),
        pltpu.VMEM((_L * _NB,), jnp.int32),
        pltpu.VMEM((_L,), jnp.int32),
        pltpu.VMEM((_CAP,), jnp.float32),
        pltpu.SemaphoreType.DMA,
        pltpu.SemaphoreType.DMA,
    ],
    compiler_params=pltpu.CompilerParams(needs_layout_passes=False),
)(_rank_body)


def kernel(y):
    return _rank_sc(y)
